# baseline (speedup 1.0000x reference)
import jax
import jax.numpy as jnp
from jax import lax
from jax.experimental import pallas as pl
from jax.experimental.pallas import tpu as pltpu
from jax.experimental.pallas import tpu_sc as plsc

ROWS = 128
COLS = 32768
N = ROWS * COLS
K = int(0.7 * N)
NC = 2
NS = 16
NW = NC * NS
LANES = 16
SHIFT = 17
NBINS = 1 << (32 - SHIFT)
HR = 256
HC = 128

ROWS_W = ROWS // NW
CCOLS = 8192
CPR = COLS // CCOLS
NCHUNK = ROWS_W * CPR
UNROLL = 16

LP = (0.0001415121753789439, 0.9954273382579881, -0.4640725804471214,
      0.21641043832781495, -0.05486285286206372)


def _hist_body(log_hbm, tgt_hbm, cnt_hbm, sum_hbm,
               lb0, lb1, tb0, tb1, hcnt, hsum, sl0, sl1, st0, st1):
    wid = lax.axis_index("s") * NC + lax.axis_index("c")
    row0 = wid * ROWS_W

    zi = jnp.zeros((LANES,), jnp.int32)
    zf = jnp.zeros((LANES,), jnp.float32)

    @plsc.parallel_loop(0, NBINS // LANES, unroll=8)
    def _zero(i):
        hcnt[pl.ds(i * LANES, LANES)] = zi
        hsum[pl.ds(i * LANES, LANES)] = zf

    lbufs = (lb0, lb1)
    tbufs = (tb0, tb1)
    lsems = (sl0, sl1)
    tsems = (st0, st1)
    ones = jnp.ones((LANES,), jnp.int32)

    def issue(ci, pi):
        sl = (pl.ds(row0 + ci // CPR, 1), pl.ds((ci % CPR) * CCOLS, CCOLS))
        pltpu.async_copy(log_hbm.at[sl], lbufs[pi], lsems[pi])
        pltpu.async_copy(tgt_hbm.at[sl], tbufs[pi], tsems[pi])

    def drain(pi):
        pltpu.make_async_copy(log_hbm.at[(pl.ds(0, 1), pl.ds(0, CCOLS))],
                              lbufs[pi], lsems[pi]).wait()
        pltpu.make_async_copy(tgt_hbm.at[(pl.ds(0, 1), pl.ds(0, CCOLS))],
                              tbufs[pi], tsems[pi]).wait()

    issue(0, 0)
    issue(1, 1)

    @pl.loop(0, NCHUNK, step=2)
    def _chunks(ci):
        for b in range(2):
            drain(b)

            lbuf = lbufs[b]
            tbuf = tbufs[b]

            @plsc.parallel_loop(0, CCOLS // LANES, unroll=UNROLL)
            def _scan(i):
                l = lbuf[0, pl.ds(i * LANES, LANES)]
                t = tbuf[0, pl.ds(i * LANES, LANES)]
                u = jnp.exp(-jnp.abs(l))
                sp = LP[4]
                for c in (LP[3], LP[2], LP[1], LP[0]):
                    sp = sp * u + c
                v = jnp.maximum(l, 0.0) - l * t + sp
                key = lax.bitcast_convert_type(v, jnp.int32)
                b2 = lax.shift_right_logical(key, SHIFT)
                plsc.addupdate_scatter(hcnt, [b2], ones)
                plsc.addupdate_scatter(hsum, [b2], v)

            @pl.when(ci + b + 2 < NCHUNK)
            def _prefetch():
                issue(ci + b + 2, b)

    pltpu.sync_copy(hcnt, cnt_hbm.at[wid])
    pltpu.sync_copy(hsum, sum_hbm.at[wid])


def _select_body(cnt_ref, sum_ref, o_ref):
    c2 = jnp.sum(cnt_ref[...].astype(jnp.float32), axis=0)
    s2 = jnp.sum(sum_ref[...], axis=0)

    p = lax.broadcasted_iota(jnp.int32, (HC, HC), 0)
    q = lax.broadcasted_iota(jnp.int32, (HC, HC), 1)
    upper = (p >= q).astype(jnp.float32)
    row_suf = jnp.dot(c2, upper, preferred_element_type=jnp.float32)
    r0 = lax.broadcasted_iota(jnp.int32, (HR, HR), 0)
    r1 = lax.broadcasted_iota(jnp.int32, (HR, HR), 1)
    strict = (r1 > r0).astype(jnp.float32)
    rows_below = jnp.dot(strict, row_suf[:, 0:1],
                         preferred_element_type=jnp.float32)
    suf = row_suf + rows_below

    idx = (lax.broadcasted_iota(jnp.int32, (HR, HC), 0) * HC
           + lax.broadcasted_iota(jnp.int32, (HR, HC), 1))
    kf = jnp.float32(K)
    b = jnp.max(jnp.where(suf >= kf, idx, -1))

    above = idx > b
    c_above = jnp.sum(jnp.where(above, c2, 0.0))
    s_above = jnp.sum(jnp.where(above, s2, 0.0))
    at_b = idx == b
    c_b = jnp.sum(jnp.where(at_b, c2, 0.0))
    s_b = jnp.sum(jnp.where(at_b, s2, 0.0))

    r_need = kf - c_above
    lo = lax.bitcast_convert_type(b << SHIFT, jnp.float32)
    hi = lax.bitcast_convert_type((b + 1) << SHIFT, jnp.float32)
    w = hi - lo
    m = c_b - r_need
    s_top_b = s_b - m * (lo + m * w / (2.0 * c_b))
    o_ref[...] = jnp.broadcast_to((s_above + s_top_b) / kf, (1, 1))


def kernel(logits, targets):
    hist = pl.kernel(
        _hist_body,
        out_type=[jax.ShapeDtypeStruct((NW, NBINS), jnp.int32),
                  jax.ShapeDtypeStruct((NW, NBINS), jnp.float32)],
        mesh=plsc.VectorSubcoreMesh(core_axis_name="c", subcore_axis_name="s"),
        compiler_params=pltpu.CompilerParams(needs_layout_passes=False),
        scratch_types=[
            pltpu.VMEM((1, CCOLS), jnp.float32),
            pltpu.VMEM((1, CCOLS), jnp.float32),
            pltpu.VMEM((1, CCOLS), jnp.float32),
            pltpu.VMEM((1, CCOLS), jnp.float32),
            pltpu.VMEM((NBINS,), jnp.int32),
            pltpu.VMEM((NBINS,), jnp.float32),
            pltpu.SemaphoreType.DMA,
            pltpu.SemaphoreType.DMA,
            pltpu.SemaphoreType.DMA,
            pltpu.SemaphoreType.DMA,
        ],
    )
    cnt, sums = hist(logits, targets)

    out = pl.pallas_call(
        _select_body,
        out_shape=jax.ShapeDtypeStruct((1, 1), jnp.float32),
    )(cnt.reshape(NW, HR, HC), sums.reshape(NW, HR, HC))
    return out.reshape(())


# speedup vs baseline: 1.2891x; 1.2891x over previous
import jax
import jax.numpy as jnp
from jax import lax
from jax.experimental import pallas as pl
from jax.experimental.pallas import tpu as pltpu
from jax.experimental.pallas import tpu_sc as plsc

ROWS = 128
COLS = 32768
N = ROWS * COLS
K = int(0.7 * N)
NC = 2
NS = 16
NW = NC * NS
LANES = 16
SHIFT = 17
NBINS = 1 << (32 - SHIFT)
HR = 256
HC = 128

ROWS_W = ROWS // NW
CCOLS = 8192
CPR = COLS // CCOLS
NCHUNK = ROWS_W * CPR
UNROLL = 8

LP = (0.0001415121753789439, 0.9954273382579881, -0.4640725804471214,
      0.21641043832781495, -0.05486285286206372)


def _hist_body(log_hbm, tgt_hbm, cnt_hbm, sum_hbm,
               lb0, lb1, tb0, tb1, hcnt, hsum, sl0, sl1, st0, st1):
    wid = lax.axis_index("s") * NC + lax.axis_index("c")
    row0 = wid * ROWS_W

    zi = jnp.zeros((LANES,), jnp.int32)
    zf = jnp.zeros((LANES,), jnp.float32)

    @plsc.parallel_loop(0, NBINS // LANES, unroll=8)
    def _zero(i):
        hcnt[pl.ds(i * LANES, LANES)] = zi
        hsum[pl.ds(i * LANES, LANES)] = zf

    lbufs = (lb0, lb1)
    tbufs = (tb0, tb1)
    lsems = (sl0, sl1)
    tsems = (st0, st1)
    ones = jnp.ones((LANES,), jnp.int32)

    def issue(ci, pi):
        sl = (pl.ds(row0 + ci // CPR, 1), pl.ds((ci % CPR) * CCOLS, CCOLS))
        pltpu.async_copy(log_hbm.at[sl], lbufs[pi], lsems[pi])
        pltpu.async_copy(tgt_hbm.at[sl], tbufs[pi], tsems[pi])

    def drain(pi):
        pltpu.make_async_copy(log_hbm.at[(pl.ds(0, 1), pl.ds(0, CCOLS))],
                              lbufs[pi], lsems[pi]).wait()
        pltpu.make_async_copy(tgt_hbm.at[(pl.ds(0, 1), pl.ds(0, CCOLS))],
                              tbufs[pi], tsems[pi]).wait()

    issue(0, 0)
    issue(1, 1)

    @pl.loop(0, NCHUNK, step=2)
    def _chunks(ci):
        for b in range(2):
            drain(b)

            lbuf = lbufs[b]
            tbuf = tbufs[b]

            @plsc.parallel_loop(0, CCOLS // LANES, unroll=UNROLL)
            def _scan(i):
                l = lbuf[0, pl.ds(i * LANES, LANES)]
                t = tbuf[0, pl.ds(i * LANES, LANES)]
                u = jnp.exp(-jnp.abs(l))
                sp = LP[4]
                for c in (LP[3], LP[2], LP[1], LP[0]):
                    sp = sp * u + c
                v = jnp.maximum(l, 0.0) - l * t + sp
                key = lax.bitcast_convert_type(v, jnp.int32)
                b2 = lax.shift_right_logical(key, SHIFT)
                plsc.addupdate_scatter(hcnt, [b2], ones)
                plsc.addupdate_scatter(hsum, [b2], v)

            @pl.when(ci + b + 2 < NCHUNK)
            def _prefetch():
                issue(ci + b + 2, b)

    pltpu.sync_copy(hcnt, cnt_hbm.at[wid])
    pltpu.sync_copy(hsum, sum_hbm.at[wid])


def _select_body(cnt_ref, sum_ref, o_ref):
    c2 = jnp.sum(cnt_ref[...].astype(jnp.float32), axis=0)
    s2 = jnp.sum(sum_ref[...], axis=0)

    p = lax.broadcasted_iota(jnp.int32, (HC, HC), 0)
    q = lax.broadcasted_iota(jnp.int32, (HC, HC), 1)
    upper = (p >= q).astype(jnp.float32)
    row_suf = jnp.dot(c2, upper, preferred_element_type=jnp.float32)
    r0 = lax.broadcasted_iota(jnp.int32, (HR, HR), 0)
    r1 = lax.broadcasted_iota(jnp.int32, (HR, HR), 1)
    strict = (r1 > r0).astype(jnp.float32)
    rows_below = jnp.dot(strict, row_suf[:, 0:1],
                         preferred_element_type=jnp.float32)
    suf = row_suf + rows_below

    idx = (lax.broadcasted_iota(jnp.int32, (HR, HC), 0) * HC
           + lax.broadcasted_iota(jnp.int32, (HR, HC), 1))
    kf = jnp.float32(K)
    b = jnp.max(jnp.where(suf >= kf, idx, -1))

    above = idx > b
    c_above = jnp.sum(jnp.where(above, c2, 0.0))
    s_above = jnp.sum(jnp.where(above, s2, 0.0))
    at_b = idx == b
    c_b = jnp.sum(jnp.where(at_b, c2, 0.0))
    s_b = jnp.sum(jnp.where(at_b, s2, 0.0))

    r_need = kf - c_above
    lo = lax.bitcast_convert_type(b << SHIFT, jnp.float32)
    hi = lax.bitcast_convert_type((b + 1) << SHIFT, jnp.float32)
    w = hi - lo
    m = c_b - r_need
    s_top_b = s_b - m * (lo + m * w / (2.0 * c_b))
    o_ref[...] = jnp.broadcast_to((s_above + s_top_b) / kf, (1, 1))


def kernel(logits, targets):
    hist = pl.kernel(
        _hist_body,
        out_type=[jax.ShapeDtypeStruct((NW, NBINS), jnp.int32),
                  jax.ShapeDtypeStruct((NW, NBINS), jnp.float32)],
        mesh=plsc.VectorSubcoreMesh(core_axis_name="c", subcore_axis_name="s"),
        compiler_params=pltpu.CompilerParams(needs_layout_passes=False),
        scratch_types=[
            pltpu.VMEM((1, CCOLS), jnp.float32),
            pltpu.VMEM((1, CCOLS), jnp.float32),
            pltpu.VMEM((1, CCOLS), jnp.float32),
            pltpu.VMEM((1, CCOLS), jnp.float32),
            pltpu.VMEM((NBINS,), jnp.int32),
            pltpu.VMEM((NBINS,), jnp.float32),
            pltpu.SemaphoreType.DMA,
            pltpu.SemaphoreType.DMA,
            pltpu.SemaphoreType.DMA,
            pltpu.SemaphoreType.DMA,
        ],
    )
    cnt, sums = hist(logits, targets)

    out = pl.pallas_call(
        _select_body,
        out_shape=jax.ShapeDtypeStruct((1, 1), jnp.float32),
    )(cnt.reshape(NW, HR, HC), sums.reshape(NW, HR, HC))
    return out.reshape(())


# speedup vs baseline: 1.3000x; 1.0085x over previous
import jax
import jax.numpy as jnp
from jax import lax
from jax.experimental import pallas as pl
from jax.experimental.pallas import tpu as pltpu
from jax.experimental.pallas import tpu_sc as plsc

ROWS = 128
COLS = 32768
N = ROWS * COLS
K = int(0.7 * N)
NC = 2
NS = 16
NW = NC * NS
LANES = 16
SHIFT = 17
NBINS = 1 << (32 - SHIFT)
HR = 256
HC = 128

ROWS_W = ROWS // NW
CCOLS = 8192
CPR = COLS // CCOLS
NCHUNK = ROWS_W * CPR
UNROLL = 8

LP = (0.0001415121753789439, 0.9954273382579881, -0.4640725804471214,
      0.21641043832781495, -0.05486285286206372)


def _hist_body(log_hbm, tgt_hbm, cnt_hbm, sum_hbm,
               lb0, lb1, tb0, tb1, hcnt, hsum, sl0, sl1, st0, st1):
    wid = lax.axis_index("s") * NC + lax.axis_index("c")
    row0 = wid * ROWS_W

    zi = jnp.zeros((LANES,), jnp.int32)
    zf = jnp.zeros((LANES,), jnp.float32)

    @plsc.parallel_loop(0, NBINS // LANES, unroll=8)
    def _zero(i):
        hcnt[pl.ds(i * LANES, LANES)] = zi
        hsum[pl.ds(i * LANES, LANES)] = zf

    lbufs = (lb0, lb1)
    tbufs = (tb0, tb1)
    lsems = (sl0, sl1)
    tsems = (st0, st1)
    ones = jnp.ones((LANES,), jnp.int32)

    def issue(ci, pi):
        sl = (pl.ds(row0 + ci // CPR, 1), pl.ds((ci % CPR) * CCOLS, CCOLS))
        pltpu.async_copy(log_hbm.at[sl], lbufs[pi], lsems[pi])
        pltpu.async_copy(tgt_hbm.at[sl], tbufs[pi], tsems[pi])

    def drain(pi):
        pltpu.make_async_copy(log_hbm.at[(pl.ds(0, 1), pl.ds(0, CCOLS))],
                              lbufs[pi], lsems[pi]).wait()
        pltpu.make_async_copy(tgt_hbm.at[(pl.ds(0, 1), pl.ds(0, CCOLS))],
                              tbufs[pi], tsems[pi]).wait()

    issue(0, 0)
    issue(1, 1)

    @pl.loop(0, NCHUNK, step=2)
    def _chunks(ci):
        for b in range(2):
            drain(b)

            lbuf = lbufs[b]
            tbuf = tbufs[b]

            @plsc.parallel_loop(0, CCOLS // LANES, unroll=UNROLL)
            def _scan(i):
                l = lbuf[0, pl.ds(i * LANES, LANES)]
                t = tbuf[0, pl.ds(i * LANES, LANES)]
                u = jnp.exp(-jnp.abs(l))
                sp = LP[4]
                for c in (LP[3], LP[2], LP[1], LP[0]):
                    sp = sp * u + c
                v = jnp.maximum(jnp.maximum(l, 0.0) - l * t + sp, 0.0)
                key = lax.bitcast_convert_type(v, jnp.int32)
                b2 = lax.shift_right_logical(key, SHIFT)
                plsc.addupdate_scatter(hcnt, [b2], ones)
                plsc.addupdate_scatter(hsum, [b2], v)

            @pl.when(ci + b + 2 < NCHUNK)
            def _prefetch():
                issue(ci + b + 2, b)

    pltpu.sync_copy(hcnt, cnt_hbm.at[wid])
    pltpu.sync_copy(hsum, sum_hbm.at[wid])


def _select_body(cnt_ref, sum_ref, o_ref):
    c2 = jnp.sum(cnt_ref[...].astype(jnp.float32), axis=0)
    s2 = jnp.sum(sum_ref[...], axis=0)

    p = lax.broadcasted_iota(jnp.int32, (HC, HC), 0)
    q = lax.broadcasted_iota(jnp.int32, (HC, HC), 1)
    upper = (p >= q).astype(jnp.float32)
    row_suf = jnp.dot(c2, upper, preferred_element_type=jnp.float32)
    r0 = lax.broadcasted_iota(jnp.int32, (HR, HR), 0)
    r1 = lax.broadcasted_iota(jnp.int32, (HR, HR), 1)
    strict = (r1 > r0).astype(jnp.float32)
    rows_below = jnp.dot(strict, row_suf[:, 0:1],
                         preferred_element_type=jnp.float32)
    suf = row_suf + rows_below

    idx = (lax.broadcasted_iota(jnp.int32, (HR, HC), 0) * HC
           + lax.broadcasted_iota(jnp.int32, (HR, HC), 1))
    kf = jnp.float32(K)
    b = jnp.max(jnp.where(suf >= kf, idx, -1))

    above = idx > b
    c_above = jnp.sum(jnp.where(above, c2, 0.0))
    s_above = jnp.sum(jnp.where(above, s2, 0.0))
    at_b = idx == b
    c_b = jnp.sum(jnp.where(at_b, c2, 0.0))
    s_b = jnp.sum(jnp.where(at_b, s2, 0.0))

    r_need = kf - c_above
    lo = lax.bitcast_convert_type(b << SHIFT, jnp.float32)
    hi = lax.bitcast_convert_type((b + 1) << SHIFT, jnp.float32)
    w = hi - lo
    m = c_b - r_need
    s_top_b = s_b - m * (lo + m * w / (2.0 * c_b))
    o_ref[...] = jnp.broadcast_to((s_above + s_top_b) / kf, (1, 1))


def kernel(logits, targets):
    hist = pl.kernel(
        _hist_body,
        out_type=[jax.ShapeDtypeStruct((NW, NBINS), jnp.int32),
                  jax.ShapeDtypeStruct((NW, NBINS), jnp.float32)],
        mesh=plsc.VectorSubcoreMesh(core_axis_name="c", subcore_axis_name="s"),
        compiler_params=pltpu.CompilerParams(needs_layout_passes=False),
        scratch_types=[
            pltpu.VMEM((1, CCOLS), jnp.float32),
            pltpu.VMEM((1, CCOLS), jnp.float32),
            pltpu.VMEM((1, CCOLS), jnp.float32),
            pltpu.VMEM((1, CCOLS), jnp.float32),
            pltpu.VMEM((NBINS,), jnp.int32),
            pltpu.VMEM((NBINS,), jnp.float32),
            pltpu.SemaphoreType.DMA,
            pltpu.SemaphoreType.DMA,
            pltpu.SemaphoreType.DMA,
            pltpu.SemaphoreType.DMA,
        ],
    )
    cnt, sums = hist(logits, targets)

    out = pl.pallas_call(
        _select_body,
        out_shape=jax.ShapeDtypeStruct((1, 1), jnp.float32),
    )(cnt.reshape(NW, HR, HC), sums.reshape(NW, HR, HC))
    return out.reshape(())
